# Initial kernel scaffold; baseline (speedup 1.0000x reference)
import jax
import jax.numpy as jnp
from jax.experimental import pallas as pl


def kernel(token_ids, weight):
    raise NotImplementedError("write your pallas kernel here")


# speedup vs baseline: 2.9667x; 2.9667x over previous
import functools

import jax
import jax.numpy as jnp
from jax import lax
from jax.experimental import pallas as pl
from jax.experimental.pallas import tpu as pltpu
from jax.experimental.pallas import tpu_sc as plsc

D = 128
CH = 128

_info = plsc.get_sparse_core_info()
NC = _info.num_cores
NS = _info.num_subcores
NW = NC * NS


def _make_gather(B: int):
    assert B % (NW * CH) == 0
    bpw = B // NW
    nchunk = bpw // CH

    mesh = plsc.VectorSubcoreMesh(core_axis_name="c", subcore_axis_name="s")

    @functools.partial(
        pl.kernel,
        out_type=jax.ShapeDtypeStruct((B, D), jnp.float32),
        mesh=mesh,
        scratch_types=[
            pltpu.VMEM((nchunk, CH), jnp.int32),
            pltpu.VMEM((CH, D), jnp.float32),
            pltpu.SemaphoreType.DMA,
        ],
    )
    def gather_kernel(table_hbm, idx_hbm, out_hbm, idx_v, rows_v, sem):
        wid = lax.axis_index("s") * NC + lax.axis_index("c")
        base = wid * bpw
        pltpu.sync_copy(idx_hbm.at[wid], idx_v)

        def chunk_body(c, carry):
            pltpu.async_copy(table_hbm.at[idx_v.at[c]], rows_v, sem).wait()
            pltpu.sync_copy(rows_v, out_hbm.at[pl.ds(base + c * CH, CH)])
            return carry

        lax.fori_loop(0, nchunk, chunk_body, 0)

    return gather_kernel


def kernel(token_ids, weight):
    shape = token_ids.shape
    B = token_ids.size
    idx = token_ids.reshape(NW, B // (NW * CH), CH).astype(jnp.int32)
    out = _make_gather(B)(weight, idx)
    return out.reshape(shape + (D,))


# speedup vs baseline: 3.3322x; 1.1232x over previous
import functools

import jax
import jax.numpy as jnp
from jax import lax
from jax.experimental import pallas as pl
from jax.experimental.pallas import tpu as pltpu
from jax.experimental.pallas import tpu_sc as plsc

D = 128
CH = 128

_info = plsc.get_sparse_core_info()
NC = _info.num_cores
NS = _info.num_subcores
NW = NC * NS


def _make_gather(B: int):
    assert B % (NW * CH) == 0
    bpw = B // NW
    nchunk = bpw // CH

    mesh = plsc.VectorSubcoreMesh(core_axis_name="c", subcore_axis_name="s")

    @functools.partial(
        pl.kernel,
        out_type=jax.ShapeDtypeStruct((B, D), jnp.float32),
        mesh=mesh,
        scratch_types=[
            pltpu.VMEM((nchunk, CH), jnp.int32),
            pltpu.VMEM((2, CH, D), jnp.float32),
            pltpu.SemaphoreType.DMA((2,)),
        ],
    )
    def gather_kernel(table_hbm, idx_hbm, out_hbm, idx_v, rows_v, sem):
        wid = lax.axis_index("s") * NC + lax.axis_index("c")
        base = wid * bpw
        pltpu.sync_copy(idx_hbm.at[wid], idx_v)

        def start_gather(c, b):
            pltpu.async_copy(table_hbm.at[idx_v.at[c]], rows_v.at[b], sem.at[b])

        def wait_gather(c, b):
            pltpu.make_async_copy(
                table_hbm.at[idx_v.at[c]], rows_v.at[b], sem.at[b]
            ).wait()

        start_gather(0, 0)

        def pair_body(p, carry):
            for b in range(2):
                c = 2 * p + b

                @pl.when(c + 1 < nchunk)
                def _():
                    start_gather(c + 1, 1 - b)

                wait_gather(c, b)
                pltpu.sync_copy(rows_v.at[b], out_hbm.at[pl.ds(base + c * CH, CH)])
            return carry

        lax.fori_loop(0, nchunk // 2, pair_body, 0)

    return gather_kernel


def kernel(token_ids, weight):
    shape = token_ids.shape
    B = token_ids.size
    idx = token_ids.reshape(NW, B // (NW * CH), CH).astype(jnp.int32)
    out = _make_gather(B)(weight, idx)
    return out.reshape(shape + (D,))


# speedup vs baseline: 3.3435x; 1.0034x over previous
import functools

import jax
import jax.numpy as jnp
from jax import lax
from jax.experimental import pallas as pl
from jax.experimental.pallas import tpu as pltpu
from jax.experimental.pallas import tpu_sc as plsc

D = 128
CH = 128

_info = plsc.get_sparse_core_info()
NC = _info.num_cores
NS = _info.num_subcores
NW = NC * NS


def _make_gather(B: int):
    assert B % (NW * CH) == 0
    bpw = B // NW
    nchunk = bpw // CH

    mesh = plsc.VectorSubcoreMesh(core_axis_name="c", subcore_axis_name="s")

    K = 5
    assert nchunk % K == 0 and nchunk > K

    @functools.partial(
        pl.kernel,
        out_type=jax.ShapeDtypeStruct((B, D), jnp.float32),
        mesh=mesh,
        scratch_types=[
            pltpu.VMEM((nchunk, CH), jnp.int32),
            pltpu.VMEM((K, CH, D), jnp.float32),
            pltpu.SemaphoreType.DMA((K,)),
            pltpu.SemaphoreType.DMA((K,)),
        ],
    )
    def gather_kernel(table_hbm, idx_hbm, out_hbm, idx_v, rows_v, gsem, ssem):
        wid = lax.axis_index("s") * NC + lax.axis_index("c")
        base = wid * bpw
        pltpu.sync_copy(idx_hbm.at[wid], idx_v)

        def gather_desc(c, b):
            return pltpu.make_async_copy(
                table_hbm.at[idx_v.at[c]], rows_v.at[b], gsem.at[b]
            )

        def store_desc(c, b):
            return pltpu.make_async_copy(
                rows_v.at[b], out_hbm.at[pl.ds(base + c * CH, CH)], ssem.at[b]
            )

        for b in range(K):
            gather_desc(b, b).start()

        def round_body(p, carry):
            for b in range(K):
                c = K * p + b
                gather_desc(c, b).wait()
                store_desc(c, b).start()
                bp = (b - 1) % K
                cp = c - 1
                cn = cp + K

                @pl.when(jnp.logical_and(cp >= 0, cn < nchunk))
                def _():
                    store_desc(cp, bp).wait()
                    gather_desc(cn, bp).start()

            return carry

        lax.fori_loop(0, nchunk // K, round_body, 0)

        for i in range(K):
            c = nchunk - K + i
            store_desc(c, c % K).wait()

    return gather_kernel


def kernel(token_ids, weight):
    shape = token_ids.shape
    B = token_ids.size
    idx = token_ids.reshape(NW, B // (NW * CH), CH).astype(jnp.int32)
    out = _make_gather(B)(weight, idx)
    return out.reshape(shape + (D,))
